# Initial kernel scaffold; baseline (speedup 1.0000x reference)
import jax
import jax.numpy as jnp
from jax.experimental import pallas as pl


def kernel(atom_state, bond_state, src_idx, dst_idx, batch_idx, bond_mask, num_atoms, W1, b1, W2, b2, U1, ub1, U2, ub2):
    raise NotImplementedError("write your pallas kernel here")


# speedup vs baseline: 10.3929x; 10.3929x over previous
import functools

import jax
import jax.numpy as jnp
from jax import lax
from jax.experimental import pallas as pl
from jax.experimental.pallas import tpu as pltpu
from jax.experimental.pallas import tpu_sc as plsc

NC = 2
NS = 16
NW = NC * NS


def _sc_gather(table, gidx, H):
    BE = gidx.shape[0] * 1024
    per_w = BE // NW
    n_blocks = per_w // 1024

    def body(table_hbm, gidx_hbm, out_hbm, idx_v, rows_v, sem):
        c = lax.axis_index("c")
        s = lax.axis_index("s")
        wid = s * NC + c
        base = wid * per_w

        def block(i, carry):
            pltpu.sync_copy(gidx_hbm.at[base // 1024 + i], idx_v)
            for h in range(2):
                cps = [
                    pltpu.async_copy(
                        table_hbm.at[idx_v.at[h * 4 + j]],
                        rows_v.at[pl.ds(j * 128, 128)],
                        sem,
                    )
                    for j in range(4)
                ]
                for cp in cps:
                    cp.wait()
                pltpu.sync_copy(
                    rows_v, out_hbm.at[pl.ds(base + i * 1024 + h * 512, 512)]
                )
            return carry

        lax.fori_loop(0, n_blocks, block, 0)

    return pl.kernel(
        body,
        out_type=jax.ShapeDtypeStruct((BE, H), jnp.float32),
        mesh=plsc.VectorSubcoreMesh(
            core_axis_name="c", subcore_axis_name="s",
            num_cores=NC, num_subcores=NS,
        ),
        scratch_types=[
            pltpu.VMEM((8, 128), jnp.int32),
            pltpu.VMEM((512, H), jnp.float32),
            pltpu.SemaphoreType.DMA,
        ],
    )(table, gidx)


def _sc_scatter(messages, sidx, zeros, B, A, E, H):
    GB = 1
    rows_g = GB * A
    per_tile_out = rows_g // NS
    edges_g = GB * E
    per_tile_e = edges_g // NS
    n_blocks = per_tile_e // 1024
    n_groups = (B // GB) // NC
    b_per_sc = B // NC

    def body(msg_hbm, sidx_hbm, zer_hbm, agg_hbm, idx_v, msg_v, acc_sh, sem):
        c = lax.axis_index("c")
        s = lax.axis_index("s")
        for g in range(n_groups):
            b0 = c * b_per_sc + g * GB
            pltpu.sync_copy(zer_hbm, acc_sh.at[pl.ds(s * per_tile_out, per_tile_out)])
            plsc.subcore_barrier()
            ebase = b0 * E + s * per_tile_e

            def block(i, carry):
                pltpu.sync_copy(sidx_hbm.at[ebase // 1024 + i], idx_v)
                for h in range(2):
                    pltpu.sync_copy(
                        msg_hbm.at[pl.ds(ebase + i * 1024 + h * 512, 512)],
                        msg_v,
                    )
                    for j in range(4):
                        pltpu.sync_copy(
                            msg_v.at[pl.ds(j * 128, 128)],
                            acc_sh.at[idx_v.at[h * 4 + j]],
                            add=True,
                        )
                return carry

            lax.fori_loop(0, n_blocks, block, 0)
            plsc.subcore_barrier()
            pltpu.sync_copy(
                acc_sh.at[pl.ds(s * per_tile_out, per_tile_out)],
                agg_hbm.at[pl.ds(b0 * A + s * per_tile_out, per_tile_out)],
            )

    return pl.kernel(
        body,
        out_type=jax.ShapeDtypeStruct((B * A, H), jnp.float32),
        mesh=plsc.VectorSubcoreMesh(
            core_axis_name="c", subcore_axis_name="s",
            num_cores=NC, num_subcores=NS,
        ),
        scratch_types=[
            pltpu.VMEM((8, 128), jnp.int32),
            pltpu.VMEM((512, H), jnp.float32),
            pltpu.VMEM_SHARED((rows_g, H), jnp.float32),
            pltpu.SemaphoreType.DMA,
        ],
    )(messages, sidx, zeros)


def _edge_mlp_body(n_ref, bd_ref, m_ref, w1_ref, b1_ref, w2_ref, b2_ref, o_ref):
    x = jnp.concatenate([n_ref[...], bd_ref[...]], axis=-1)
    x = jnp.dot(x, w1_ref[...], preferred_element_type=jnp.float32) + b1_ref[...]
    x = jnp.maximum(x, 0.0)
    y = jnp.dot(x, w2_ref[...], preferred_element_type=jnp.float32) + b2_ref[...]
    o_ref[...] = y * m_ref[...]


def _tc_edge_mlp(neighbor, bond, mask, W1, b1r, W2, b2r):
    BE, H = neighbor.shape
    RB = 2048
    grid = (BE // RB,)
    full = lambda shape: pl.BlockSpec(shape, lambda i: (0, 0))
    return pl.pallas_call(
        _edge_mlp_body,
        grid=grid,
        in_specs=[
            pl.BlockSpec((RB, H), lambda i: (i, 0)),
            pl.BlockSpec((RB, H), lambda i: (i, 0)),
            pl.BlockSpec((RB, 1), lambda i: (i, 0)),
            full(W1.shape),
            full(b1r.shape),
            full(W2.shape),
            full(b2r.shape),
        ],
        out_specs=pl.BlockSpec((RB, H), lambda i: (i, 0)),
        out_shape=jax.ShapeDtypeStruct((BE, H), jnp.float32),
    )(neighbor, bond, mask, W1, b1r, W2, b2r)


def _node_mlp_body(a_ref, u1_ref, ub1_ref, u2_ref, ub2_ref, o_ref):
    h = jnp.dot(a_ref[...], u1_ref[...], preferred_element_type=jnp.float32) + ub1_ref[...]
    h = jnp.maximum(h, 0.0)
    o_ref[...] = jnp.dot(h, u2_ref[...], preferred_element_type=jnp.float32) + ub2_ref[...]


def _tc_node_mlp(agg, U1, ub1r, U2, ub2r):
    N, H = agg.shape
    RB = 2048
    grid = (N // RB,)
    full = lambda shape: pl.BlockSpec(shape, lambda i: (0, 0))
    return pl.pallas_call(
        _node_mlp_body,
        grid=grid,
        in_specs=[
            pl.BlockSpec((RB, H), lambda i: (i, 0)),
            full(U1.shape),
            full(ub1r.shape),
            full(U2.shape),
            full(ub2r.shape),
        ],
        out_specs=pl.BlockSpec((RB, H), lambda i: (i, 0)),
        out_shape=jax.ShapeDtypeStruct((N, H), jnp.float32),
    )(agg, U1, ub1r, U2, ub2r)


def kernel(atom_state, bond_state, src_idx, dst_idx, batch_idx, bond_mask,
           num_atoms, W1, b1, W2, b2, U1, ub1, U2, ub2):
    B, A, H = atom_state.shape
    E = bond_state.shape[1]
    BE = B * E

    table = atom_state.reshape(B * A, H)
    gidx = (batch_idx.astype(jnp.int32) * A + dst_idx.astype(jnp.int32))
    gidx = gidx.reshape(BE // 1024, 8, 128)
    bond2 = bond_state.reshape(BE, H)
    mask2 = bond_mask.reshape(BE, 1)
    sidx = src_idx.astype(jnp.int32).reshape(BE // 1024, 8, 128)
    zeros = jnp.zeros((A // NS, H), jnp.float32)

    neighbor = _sc_gather(table, gidx, H)
    messages = _tc_edge_mlp(
        neighbor, bond2, mask2, W1, b1.reshape(1, -1), W2, b2.reshape(1, -1)
    )
    agg = _sc_scatter(messages, sidx, zeros, B, A, E, H)
    new_atom = _tc_node_mlp(
        agg, U1, ub1.reshape(1, -1), U2, ub2.reshape(1, -1)
    )
    return new_atom.reshape(B, A, H)
